# Initial kernel scaffold; baseline (speedup 1.0000x reference)
import jax
import jax.numpy as jnp
from jax.experimental import pallas as pl


def kernel(x, table):
    raise NotImplementedError("write your pallas kernel here")


# speedup vs baseline: 7.7689x; 7.7689x over previous
import functools

import jax
import jax.numpy as jnp
from jax import lax
from jax.experimental import pallas as pl
from jax.experimental.pallas import tpu as pltpu
from jax.experimental.pallas import tpu_sc as plsc

_B, _L, _D = 16384, 200, 64
_OFFS = tuple(range(0, 192, 16)) + (184,)
_XCHUNK = 8


@functools.lru_cache(maxsize=1)
def _build():
    info = plsc.get_sparse_core_info()
    nw = info.num_cores * info.num_subcores
    rows_per_w = _B // nw
    n_chunks = rows_per_w // _XCHUNK

    mesh = plsc.VectorSubcoreMesh(core_axis_name="c", subcore_axis_name="s")

    @functools.partial(
        pl.kernel,
        out_type=jax.ShapeDtypeStruct((_B, _D, _L), jnp.float32),
        mesh=mesh,
        scratch_types=[
            pltpu.VMEM((_D * 16,), jnp.float32),
            pltpu.VMEM((_D * 16,), jnp.float32),
            pltpu.VMEM((_XCHUNK, _L), jnp.int32),
            pltpu.VMEM((_D, _L), jnp.float32),
        ],
    )
    def k(x_hbm, r0s_hbm, drs_hbm, out_hbm, r0s_v, drs_v, xc_v, obuf_v):
        c = lax.axis_index("c")
        s = lax.axis_index("s")
        wid = s * info.num_cores + c
        base = wid * rows_per_w

        pltpu.sync_copy(r0s_hbm, r0s_v)
        pltpu.sync_copy(drs_hbm, drs_v)

        def chunk_body(ci, carry):
            row0 = base + ci * _XCHUNK
            pltpu.sync_copy(x_hbm.at[pl.ds(row0, _XCHUNK)], xc_v)
            for j in range(_XCHUNK):
                bits = [
                    jnp.maximum(1 - jnp.abs(xc_v[j, pl.ds(o, 16)] - 1), 0)
                    .astype(jnp.float32)
                    for o in _OFFS
                ]

                def d_body(d, bits):
                    r0 = r0s_v[pl.ds(d * 16, 16)]
                    dr = drs_v[pl.ds(d * 16, 16)]
                    for o, bv in zip(_OFFS, bits):
                        obuf_v[d, pl.ds(o, 16)] = bv * dr + r0
                    return bits

                lax.fori_loop(0, _D, d_body, bits, unroll=False)
                pltpu.sync_copy(obuf_v, out_hbm.at[row0 + j])
            return carry

        lax.fori_loop(0, n_chunks, chunk_body, 0, unroll=False)

    return k


def kernel(x, table):
    t0 = table[0]
    r0s = jnp.repeat(t0, 16)
    drs = jnp.repeat(table[1] - t0, 16)
    return _build()(x, r0s, drs)


# speedup vs baseline: 9.9202x; 1.2769x over previous
import functools

import jax
import jax.numpy as jnp
from jax import lax
from jax.experimental import pallas as pl
from jax.experimental.pallas import tpu as pltpu
from jax.experimental.pallas import tpu_sc as plsc

_B, _L, _D = 16384, 200, 64
_OFFS = tuple(range(0, 192, 16)) + (184,)
_XCHUNK = 8


@functools.lru_cache(maxsize=1)
def _build():
    info = plsc.get_sparse_core_info()
    nw = info.num_cores * info.num_subcores
    rows_per_w = _B // nw
    n_chunks = rows_per_w // _XCHUNK

    mesh = plsc.VectorSubcoreMesh(core_axis_name="c", subcore_axis_name="s")

    @functools.partial(
        pl.kernel,
        out_type=jax.ShapeDtypeStruct((_B, _D, _L), jnp.float32),
        mesh=mesh,
        scratch_types=[
            pltpu.VMEM((_D * 16,), jnp.float32),
            pltpu.VMEM((_D * 16,), jnp.float32),
            pltpu.VMEM((2, _XCHUNK, _L), jnp.int32),
            pltpu.VMEM((2, _D, _L), jnp.float32),
            pltpu.SemaphoreType.DMA,
            pltpu.SemaphoreType.DMA,
            pltpu.SemaphoreType.DMA,
            pltpu.SemaphoreType.DMA,
        ],
    )
    def k(x_hbm, r0s_hbm, drs_hbm, out_hbm,
          r0s_v, drs_v, xc_v, obuf_v, sx0, sx1, so0, so1):
        c = lax.axis_index("c")
        s = lax.axis_index("s")
        wid = s * info.num_cores + c
        base = wid * rows_per_w
        xsems = (sx0, sx1)
        osems = (so0, so1)

        pltpu.sync_copy(r0s_hbm, r0s_v)
        pltpu.sync_copy(drs_hbm, drs_v)

        def x_start(ci, xb):
            pltpu.async_copy(
                x_hbm.at[pl.ds(base + ci * _XCHUNK, _XCHUNK)],
                xc_v.at[xb], xsems[xb])

        def x_wait(ci, xb):
            pltpu.make_async_copy(
                x_hbm.at[pl.ds(base + ci * _XCHUNK, _XCHUNK)],
                xc_v.at[xb], xsems[xb]).wait()

        def row_body(row, j, xb, skip_wait):
            p = j & 1
            if not skip_wait:
                pltpu.make_async_copy(
                    obuf_v.at[p], out_hbm.at[row - 2], osems[p]).wait()
            bits = [
                jnp.maximum(1 - jnp.abs(xc_v[xb, j, pl.ds(o, 16)] - 1), 0)
                .astype(jnp.float32)
                for o in _OFFS
            ]

            def d_body(d, bits):
                r0 = r0s_v[pl.ds(d * 16, 16)]
                dr = drs_v[pl.ds(d * 16, 16)]
                for o, bv in zip(_OFFS, bits):
                    obuf_v[p, d, pl.ds(o, 16)] = bv * dr + r0
                return bits

            lax.fori_loop(0, _D, d_body, bits, unroll=False)
            pltpu.async_copy(obuf_v.at[p], out_hbm.at[row], osems[p])

        def chunk_body(ci, xb, first, guard_prefetch):
            row0 = base + ci * _XCHUNK
            x_wait(ci, xb)
            if guard_prefetch:
                @pl.when(ci + 1 < n_chunks)
                def _():
                    x_start(ci + 1, 1 - xb)
            else:
                x_start(ci + 1, 1 - xb)
            for j in range(_XCHUNK):
                row_body(row0 + j, j, xb, skip_wait=(first and j < 2))

        x_start(0, 0)
        chunk_body(0, 0, first=True, guard_prefetch=False)
        chunk_body(1, 1, first=False, guard_prefetch=False)

        def pair_body(kk, carry):
            chunk_body(2 * kk, 0, first=False, guard_prefetch=False)
            chunk_body(2 * kk + 1, 1, first=False, guard_prefetch=True)
            return carry

        lax.fori_loop(1, n_chunks // 2, pair_body, 0, unroll=False)

        last = base + rows_per_w
        pltpu.make_async_copy(
            obuf_v.at[0], out_hbm.at[last - 2], osems[0]).wait()
        pltpu.make_async_copy(
            obuf_v.at[1], out_hbm.at[last - 1], osems[1]).wait()

    return k


def kernel(x, table):
    t0 = table[0]
    r0s = jnp.repeat(t0, 16)
    drs = jnp.repeat(table[1] - t0, 16)
    return _build()(x, r0s, drs)
